# baseline (speedup 1.0000x reference)
import functools

import jax
import jax.numpy as jnp
from jax import lax
from jax.experimental import pallas as pl
from jax.experimental.pallas import tpu as pltpu

_S = 64.0
_M = 0.4

_ROWS = 1024
_COLS = 100000
_BAND = 8
_NCHUNK = _ROWS // _BAND
_NBUF = 4


def _body(lbl_ref, cos_hbm, out_hbm, in_buf, out_buf, in_sems, out_sems):
    for b in range(_NBUF):
        pltpu.make_async_copy(
            cos_hbm.at[pl.ds(b * _BAND, _BAND), :],
            in_buf.at[b], in_sems.at[b]).start()

    cols = jax.lax.broadcasted_iota(jnp.int32, (_BAND, _COLS), 1)

    def round_step(g, _):
        for b in range(_NBUF):
            t = g * _NBUF + b
            r0 = t * _BAND
            pltpu.make_async_copy(
                cos_hbm.at[pl.ds(r0, _BAND), :],
                in_buf.at[b], in_sems.at[b]).wait()

            @pl.when(g > 0)
            def _():
                pltpu.make_async_copy(
                    out_buf.at[b],
                    out_hbm.at[pl.ds(r0, _BAND), :],
                    out_sems.at[b]).wait()

            lbl = lbl_ref[pl.ds(r0, _BAND), :]
            margin = jnp.where(cols == lbl, -_S * _M, 0.0)
            out_buf[b, :, :] = in_buf[b, :, :] * _S + margin

            pltpu.make_async_copy(
                out_buf.at[b],
                out_hbm.at[pl.ds(r0, _BAND), :],
                out_sems.at[b]).start()

            @pl.when(t + _NBUF < _NCHUNK)
            def _():
                pltpu.make_async_copy(
                    cos_hbm.at[pl.ds(r0 + _NBUF * _BAND, _BAND), :],
                    in_buf.at[b], in_sems.at[b]).start()
        return 0

    lax.fori_loop(0, _NCHUNK // _NBUF, round_step, 0)

    for b in range(_NBUF):
        pltpu.make_async_copy(
            out_buf.at[b],
            out_hbm.at[pl.ds(0, _BAND), :],
            out_sems.at[b]).wait()


@jax.jit
def kernel(cosine, label):
    rows, n_cols = cosine.shape
    return pl.pallas_call(
        _body,
        grid=(1,),
        in_specs=[
            pl.BlockSpec((rows, 1), lambda i: (0, 0)),
            pl.BlockSpec(memory_space=pltpu.MemorySpace.HBM),
        ],
        out_specs=pl.BlockSpec(memory_space=pltpu.MemorySpace.HBM),
        out_shape=jax.ShapeDtypeStruct((rows, n_cols), cosine.dtype),
        scratch_shapes=[
            pltpu.VMEM((_NBUF, _BAND, n_cols), cosine.dtype),
            pltpu.VMEM((_NBUF, _BAND, n_cols), cosine.dtype),
            pltpu.SemaphoreType.DMA((_NBUF,)),
            pltpu.SemaphoreType.DMA((_NBUF,)),
        ],
    )(label.reshape(rows, 1), cosine)
